# baseline (speedup 1.0000x reference)
import functools

import jax
import jax.numpy as jnp
from jax import lax
from jax.experimental import pallas as pl
from jax.experimental.pallas import tpu as pltpu
from jax.experimental.pallas import tpu_sc as plsc

_NI = 16384
_NJ = 200
_EMB = 32
_JB = _NJ // 8
_IB = _NI // 128
_JS = 4
_N_UNITS = _JB * _IB * 2
_NUM_WORKERS = 32
_PER_WORKER = _N_UNITS // _NUM_WORKERS
_N_OUTER = _PER_WORKER // 2


def _make_sc_lookup():
    mesh = plsc.VectorSubcoreMesh(core_axis_name="c", subcore_axis_name="s")

    scratch = (
        [pltpu.VMEM((_JS, 128), jnp.int32) for _ in range(2)]
        + [pltpu.VMEM((_JS * 128, _EMB), jnp.float32) for _ in range(2)]
        + [pltpu.VMEM((_JS, 4, 8, 128), jnp.float32) for _ in range(2)]
        + [pltpu.SemaphoreType.DMA for _ in range(6)]
    )

    @functools.partial(
        pl.kernel,
        mesh=mesh,
        out_type=jax.ShapeDtypeStruct((_NJ, 4, _IB, 8, 128), jnp.float32),
        scratch_types=scratch,
        compiler_params=pltpu.CompilerParams(
            use_tc_tiling_on_sc=False, needs_layout_passes=False
        ),
    )
    def emb_kernel(idx_hbm, table_hbm, out_hbm, *scr):
        idx_v = scr[0:2]
        rows_v = scr[2:4]
        stage_v = scr[4:6]
        si = scr[6:8]
        sg = scr[8:10]
        so = scr[10:12]

        wid = lax.axis_index("s") * 2 + lax.axis_index("c")
        u0 = wid * _PER_WORKER
        iota16 = lax.iota(jnp.int32, 16)

        def unit_coords(u):
            jh = lax.rem(u, 2)
            ib = lax.rem(lax.div(u, 2), _IB)
            jb = lax.div(u, 2 * _IB)
            return jb, ib, jh

        def start_idx(u, b):
            jb, ib, jh = unit_coords(u)
            pltpu.async_copy(
                idx_hbm.at[jb, ib, pl.ds(jh * _JS, _JS), :], idx_v[b], si[b]
            )

        def wait_idx(b):
            pltpu.make_async_copy(
                idx_hbm.at[0, 0, pl.ds(0, _JS), :], idx_v[b], si[b]
            ).wait()

        def start_gathers(b):
            for ji in range(_JS):
                pltpu.async_copy(
                    table_hbm.at[idx_v[b].at[ji]],
                    rows_v[b].at[pl.ds(ji * 128, 128)],
                    sg[b],
                )

        def wait_gathers(b):
            for ji in range(_JS):
                pltpu.make_async_copy(
                    table_hbm.at[idx_v[b].at[ji]],
                    rows_v[b].at[pl.ds(ji * 128, 128)],
                    sg[b],
                ).wait()

        def start_out(u, b):
            jb, ib, jh = unit_coords(u)
            j0 = jb * 8 + jh * _JS
            pltpu.async_copy(
                stage_v[b], out_hbm.at[pl.ds(j0, _JS), :, ib, :, :], so[b]
            )

        def wait_out(b):
            pltpu.make_async_copy(
                stage_v[b], out_hbm.at[pl.ds(0, _JS), :, 0, :, :], so[b]
            ).wait()

        def transpose_unit(b):
            def tbody(t, _):
                ji = lax.div(t, 8)
                iig = lax.rem(t, 8)
                row_vec = iota16 + (ji * 128 + iig * 16)
                for db in range(4):
                    for di in range(8):
                        col_vec = jnp.full((16,), db * 8 + di, jnp.int32)
                        v = plsc.load_gather(rows_v[b], [row_vec, col_vec])
                        stage_v[b][ji, db, di, pl.ds(iig * 16, 16)] = v
                return ()

            lax.fori_loop(0, _JS * 8, tbody, (), unroll=False)

        def step(u, b, prefetch, launch_next, wait_prev_out):
            wait_gathers(b)
            b1 = 1 - b
            if launch_next:
                wait_idx(b1)
                start_gathers(b1)
            if prefetch:
                start_idx(u + 2, b)
            if wait_prev_out:
                wait_out(b)
            transpose_unit(b)
            start_out(u, b)

        jb0, ib0, jh0 = unit_coords(u0)
        pltpu.sync_copy(
            idx_hbm.at[jb0, ib0, pl.ds(jh0 * _JS, _JS), :], idx_v[0]
        )
        start_gathers(0)
        start_idx(u0 + 1, 1)

        step(u0 + 0, 0, prefetch=True, launch_next=True, wait_prev_out=False)
        step(u0 + 1, 1, prefetch=True, launch_next=True, wait_prev_out=False)

        def outer(g, _):
            u = u0 + g * 2
            step(u, 0, prefetch=True, launch_next=True, wait_prev_out=True)
            step(u + 1, 1, prefetch=True, launch_next=True,
                 wait_prev_out=True)
            return ()

        lax.fori_loop(1, _N_OUTER - 1, outer, (), unroll=False)

        u = u0 + (_N_OUTER - 1) * 2
        step(u, 0, prefetch=False, launch_next=True, wait_prev_out=True)
        step(u + 1, 1, prefetch=False, launch_next=False,
             wait_prev_out=True)

        wait_out(0)
        wait_out(1)

    return emb_kernel


def kernel(indices, table):
    idx4 = (
        indices.astype(jnp.int32)
        .T.reshape(_JB, 8, _IB, 128)
        .transpose(0, 2, 1, 3)
    )
    out5 = _make_sc_lookup()(idx4, table)
    return out5.transpose(2, 4, 0, 1, 3).reshape(_NI, _NJ, _EMB)


# speedup vs baseline: 2.2187x; 2.2187x over previous
import functools

import jax
import jax.numpy as jnp
from jax import lax
from jax.experimental import pallas as pl
from jax.experimental.pallas import tpu as pltpu
from jax.experimental.pallas import tpu_sc as plsc

_NI = 16384
_NJ = 200
_EMB = 32
_JB = _NJ // 8
_IB = _NI // 128
_JS = 4
_N_UNITS = _JB * _IB * 2
_NUM_WORKERS = 32
_PER_WORKER = _N_UNITS // _NUM_WORKERS
_N_OUTER = _PER_WORKER // 2


def _make_sc_lookup():
    mesh = plsc.VectorSubcoreMesh(core_axis_name="c", subcore_axis_name="s")

    _PITCH = 515
    scratch = (
        [pltpu.VMEM((_JS, 128), jnp.int32) for _ in range(2)]
        + [pltpu.VMEM((_JS * 128, _EMB), jnp.float32) for _ in range(2)]
        + [pltpu.VMEM((_EMB, _PITCH), jnp.float32) for _ in range(2)]
        + [pltpu.SemaphoreType.DMA for _ in range(6)]
    )

    @functools.partial(
        pl.kernel,
        mesh=mesh,
        out_type=jax.ShapeDtypeStruct((_NJ, 4, _IB, 8, 128), jnp.float32),
        scratch_types=scratch,
        compiler_params=pltpu.CompilerParams(
            use_tc_tiling_on_sc=False, needs_layout_passes=False
        ),
    )
    def emb_kernel(idx_hbm, table_hbm, out_hbm, *scr):
        idx_v = scr[0:2]
        rows_v = scr[2:4]
        stage_v = scr[4:6]
        si = scr[6:8]
        sg = scr[8:10]
        so = scr[10:12]

        wid = lax.axis_index("s") * 2 + lax.axis_index("c")
        u0 = wid * _PER_WORKER
        iota16 = lax.iota(jnp.int32, 16)

        def unit_coords(u):
            jh = lax.rem(u, 2)
            ib = lax.rem(lax.div(u, 2), _IB)
            jb = lax.div(u, 2 * _IB)
            return jb, ib, jh

        def start_idx(u, b):
            jb, ib, jh = unit_coords(u)
            pltpu.async_copy(
                idx_hbm.at[jb, ib, pl.ds(jh * _JS, _JS), :], idx_v[b], si[b]
            )

        def wait_idx(b):
            pltpu.make_async_copy(
                idx_hbm.at[0, 0, pl.ds(0, _JS), :], idx_v[b], si[b]
            ).wait()

        def start_gathers(b):
            for ji in range(_JS):
                pltpu.async_copy(
                    table_hbm.at[idx_v[b].at[ji]],
                    rows_v[b].at[pl.ds(ji * 128, 128)],
                    sg[b],
                )

        def wait_gathers(b):
            for ji in range(_JS):
                pltpu.make_async_copy(
                    table_hbm.at[idx_v[b].at[ji]],
                    rows_v[b].at[pl.ds(ji * 128, 128)],
                    sg[b],
                ).wait()

        def start_out(u, b):
            jb, ib, jh = unit_coords(u)
            j0 = jb * 8 + jh * _JS
            for ji in range(_JS):
                for db in range(4):
                    pltpu.async_copy(
                        stage_v[b].at[pl.ds(db * 8, 8), pl.ds(ji * 128, 128)],
                        out_hbm.at[j0 + ji, db, ib, :, :],
                        so[b],
                    )

        def wait_out(b):
            for _ in range(_JS * 4):
                pltpu.make_async_copy(
                    stage_v[b].at[pl.ds(0, 8), pl.ds(0, 128)],
                    out_hbm.at[0, 0, 0, :, :],
                    so[b],
                ).wait()

        def transpose_unit(b):
            d_vecs = [iota16 + h * 16 for h in range(2)]

            def tbody(ii, _):
                for ji in range(_JS):
                    r = ji * 128 + ii
                    col_vec = jnp.full((16,), r, jnp.int32)
                    for h in range(2):
                        v = rows_v[b][r, pl.ds(h * 16, 16)]
                        plsc.store_scatter(
                            stage_v[b], [d_vecs[h], col_vec], v
                        )
                return ()

            lax.fori_loop(0, 128, tbody, (), unroll=False)

        def step(u, b, prefetch, launch_next, wait_prev_out):
            wait_gathers(b)
            b1 = 1 - b
            if launch_next:
                wait_idx(b1)
                start_gathers(b1)
            if prefetch:
                start_idx(u + 2, b)
            if wait_prev_out:
                wait_out(b)
            transpose_unit(b)
            start_out(u, b)

        jb0, ib0, jh0 = unit_coords(u0)
        pltpu.sync_copy(
            idx_hbm.at[jb0, ib0, pl.ds(jh0 * _JS, _JS), :], idx_v[0]
        )
        start_gathers(0)
        start_idx(u0 + 1, 1)

        step(u0 + 0, 0, prefetch=True, launch_next=True, wait_prev_out=False)
        step(u0 + 1, 1, prefetch=True, launch_next=True, wait_prev_out=False)

        def outer(g, _):
            u = u0 + g * 2
            step(u, 0, prefetch=True, launch_next=True, wait_prev_out=True)
            step(u + 1, 1, prefetch=True, launch_next=True,
                 wait_prev_out=True)
            return ()

        lax.fori_loop(1, _N_OUTER - 1, outer, (), unroll=False)

        u = u0 + (_N_OUTER - 1) * 2
        step(u, 0, prefetch=False, launch_next=True, wait_prev_out=True)
        step(u + 1, 1, prefetch=False, launch_next=False,
             wait_prev_out=True)

        wait_out(0)
        wait_out(1)

    return emb_kernel


def kernel(indices, table):
    idx4 = (
        indices.astype(jnp.int32)
        .T.reshape(_JB, 8, _IB, 128)
        .transpose(0, 2, 1, 3)
    )
    out5 = _make_sc_lookup()(idx4, table)
    return out5.transpose(2, 4, 0, 1, 3).reshape(_NI, _NJ, _EMB)


# speedup vs baseline: 2.2575x; 1.0175x over previous
import functools

import jax
import jax.numpy as jnp
from jax import lax
from jax.experimental import pallas as pl
from jax.experimental.pallas import tpu as pltpu
from jax.experimental.pallas import tpu_sc as plsc

_NI = 16384
_NJ = 200
_EMB = 32
_JB = _NJ // 8
_IB = _NI // 128
_JS = 4
_N_UNITS = _JB * _IB * 2
_NUM_WORKERS = 32
_PER_WORKER = _N_UNITS // _NUM_WORKERS
_N_OUTER = _PER_WORKER // 2


def _make_sc_lookup():
    mesh = plsc.VectorSubcoreMesh(core_axis_name="c", subcore_axis_name="s")

    _PITCH = 515
    scratch = (
        [pltpu.VMEM((_JS * 128,), jnp.int32) for _ in range(2)]
        + [pltpu.VMEM((_JS * 128, _EMB), jnp.float32) for _ in range(2)]
        + [pltpu.VMEM((_EMB, _PITCH), jnp.float32) for _ in range(2)]
        + [pltpu.SemaphoreType.DMA for _ in range(6)]
    )

    @functools.partial(
        pl.kernel,
        mesh=mesh,
        out_type=jax.ShapeDtypeStruct((_NJ, 4, _IB, 8, 128), jnp.float32),
        scratch_types=scratch,
        compiler_params=pltpu.CompilerParams(
            use_tc_tiling_on_sc=False, needs_layout_passes=False
        ),
    )
    def emb_kernel(idx_hbm, table_hbm, out_hbm, *scr):
        idx_v = scr[0:2]
        rows_v = scr[2:4]
        stage_v = scr[4:6]
        si = scr[6:8]
        sg = scr[8:10]
        so = scr[10:12]

        wid = lax.axis_index("s") * 2 + lax.axis_index("c")
        u0 = wid * _PER_WORKER
        iota16 = lax.iota(jnp.int32, 16)

        def unit_coords(u):
            jh = lax.rem(u, 2)
            ib = lax.rem(lax.div(u, 2), _IB)
            jb = lax.div(u, 2 * _IB)
            return jb, ib, jh

        def start_idx(u, b):
            jb, ib, jh = unit_coords(u)
            pltpu.async_copy(
                idx_hbm.at[jb, ib, pl.ds(jh * _JS * 128, _JS * 128)],
                idx_v[b],
                si[b],
            )

        def wait_idx(b):
            pltpu.make_async_copy(
                idx_hbm.at[0, 0, pl.ds(0, _JS * 128)], idx_v[b], si[b]
            ).wait()

        def start_gathers(b):
            pltpu.async_copy(table_hbm.at[idx_v[b]], rows_v[b], sg[b])

        def wait_gathers(b):
            pltpu.make_async_copy(
                table_hbm.at[idx_v[b]], rows_v[b], sg[b]
            ).wait()

        def start_out(u, b):
            jb, ib, jh = unit_coords(u)
            j0 = jb * 8 + jh * _JS
            for ji in range(_JS):
                for db in range(4):
                    pltpu.async_copy(
                        stage_v[b].at[pl.ds(db * 8, 8), pl.ds(ji * 128, 128)],
                        out_hbm.at[j0 + ji, db, ib, :, :],
                        so[b],
                    )

        def wait_out(b):
            for _ in range(_JS * 4):
                pltpu.make_async_copy(
                    stage_v[b].at[pl.ds(0, 8), pl.ds(0, 128)],
                    out_hbm.at[0, 0, 0, :, :],
                    so[b],
                ).wait()

        def transpose_unit(b):
            d_vecs = [iota16 + h * 16 for h in range(2)]

            def tbody(ii, _):
                for ji in range(_JS):
                    r = ji * 128 + ii
                    col_vec = jnp.full((16,), r, jnp.int32)
                    for h in range(2):
                        v = rows_v[b][r, pl.ds(h * 16, 16)]
                        plsc.store_scatter(
                            stage_v[b], [d_vecs[h], col_vec], v
                        )
                return ()

            lax.fori_loop(0, 128, tbody, (), unroll=4)

        def step(u, b, prefetch, launch_next, wait_prev_out):
            wait_gathers(b)
            b1 = 1 - b
            if launch_next:
                wait_idx(b1)
                start_gathers(b1)
            if prefetch:
                start_idx(u + 2, b)
            if wait_prev_out:
                wait_out(b)
            transpose_unit(b)
            start_out(u, b)

        jb0, ib0, jh0 = unit_coords(u0)
        pltpu.sync_copy(
            idx_hbm.at[jb0, ib0, pl.ds(jh0 * _JS * 128, _JS * 128)], idx_v[0]
        )
        start_gathers(0)
        start_idx(u0 + 1, 1)

        step(u0 + 0, 0, prefetch=True, launch_next=True, wait_prev_out=False)
        step(u0 + 1, 1, prefetch=True, launch_next=True, wait_prev_out=False)

        def outer(g, _):
            u = u0 + g * 2
            step(u, 0, prefetch=True, launch_next=True, wait_prev_out=True)
            step(u + 1, 1, prefetch=True, launch_next=True,
                 wait_prev_out=True)
            return ()

        lax.fori_loop(1, _N_OUTER - 1, outer, (), unroll=False)

        u = u0 + (_N_OUTER - 1) * 2
        step(u, 0, prefetch=False, launch_next=True, wait_prev_out=True)
        step(u + 1, 1, prefetch=False, launch_next=False,
             wait_prev_out=True)

        wait_out(0)
        wait_out(1)

    return emb_kernel


def kernel(indices, table):
    idx4 = (
        indices.astype(jnp.int32)
        .T.reshape(_JB, 8, _IB, 128)
        .transpose(0, 2, 1, 3)
        .reshape(_JB, _IB, 8 * 128)
    )
    out5 = _make_sc_lookup()(idx4, table)
    return out5.transpose(2, 4, 0, 1, 3).reshape(_NI, _NJ, _EMB)


# speedup vs baseline: 2.2712x; 1.0061x over previous
import functools

import jax
import jax.numpy as jnp
from jax import lax
from jax.experimental import pallas as pl
from jax.experimental.pallas import tpu as pltpu
from jax.experimental.pallas import tpu_sc as plsc

_NI = 16384
_NJ = 200
_EMB = 32
_JB = _NJ // 8
_IB = _NI // 128
_JS = 4
_N_UNITS = _JB * _IB * 2
_NUM_WORKERS = 32
_PER_WORKER = _N_UNITS // _NUM_WORKERS
_N_OUTER = _PER_WORKER // 2


def _make_sc_lookup():
    mesh = plsc.VectorSubcoreMesh(core_axis_name="c", subcore_axis_name="s")

    _PITCH = 515
    scratch = (
        [pltpu.VMEM((_JS * 128,), jnp.int32) for _ in range(2)]
        + [pltpu.VMEM((_JS * 128, _EMB), jnp.float32) for _ in range(2)]
        + [pltpu.VMEM((4, 8, _PITCH), jnp.float32) for _ in range(2)]
        + [pltpu.SemaphoreType.DMA for _ in range(6)]
    )

    @functools.partial(
        pl.kernel,
        mesh=mesh,
        out_type=jax.ShapeDtypeStruct((_NJ, 4, _IB, 8, 128), jnp.float32),
        scratch_types=scratch,
        compiler_params=pltpu.CompilerParams(
            use_tc_tiling_on_sc=False, needs_layout_passes=False
        ),
    )
    def emb_kernel(idx_hbm, table_hbm, out_hbm, *scr):
        idx_v = scr[0:2]
        rows_v = scr[2:4]
        stage_v = scr[4:6]
        si = scr[6:8]
        sg = scr[8:10]
        so = scr[10:12]

        wid = lax.axis_index("s") * 2 + lax.axis_index("c")
        u0 = wid * _PER_WORKER
        iota16 = lax.iota(jnp.int32, 16)

        def unit_coords(u):
            jh = lax.rem(u, 2)
            ib = lax.rem(lax.div(u, 2), _IB)
            jb = lax.div(u, 2 * _IB)
            return jb, ib, jh

        def start_idx(u, b):
            jb, ib, jh = unit_coords(u)
            pltpu.async_copy(
                idx_hbm.at[jb, ib, pl.ds(jh * _JS * 128, _JS * 128)],
                idx_v[b],
                si[b],
            )

        def wait_idx(b):
            pltpu.make_async_copy(
                idx_hbm.at[0, 0, pl.ds(0, _JS * 128)], idx_v[b], si[b]
            ).wait()

        def start_gathers(b):
            pltpu.async_copy(table_hbm.at[idx_v[b]], rows_v[b], sg[b])

        def wait_gathers(b):
            pltpu.make_async_copy(
                table_hbm.at[idx_v[b]], rows_v[b], sg[b]
            ).wait()

        def start_out(u, b):
            jb, ib, jh = unit_coords(u)
            j0 = jb * 8 + jh * _JS
            for ji in range(_JS):
                pltpu.async_copy(
                    stage_v[b].at[:, :, pl.ds(ji * 128, 128)],
                    out_hbm.at[j0 + ji, :, ib, :, :],
                    so[b],
                )

        def wait_out(b):
            for _ in range(_JS):
                pltpu.make_async_copy(
                    stage_v[b].at[:, :, pl.ds(0, 128)],
                    out_hbm.at[0, :, 0, :, :],
                    so[b],
                ).wait()

        def transpose_unit(b):
            db_vecs = [(iota16 + h * 16) // 8 for h in range(2)]
            di_vecs = [(iota16 + h * 16) % 8 for h in range(2)]

            def tbody(ii, _):
                for ji in range(_JS):
                    r = ji * 128 + ii
                    col_vec = jnp.full((16,), r, jnp.int32)
                    for h in range(2):
                        v = rows_v[b][r, pl.ds(h * 16, 16)]
                        plsc.store_scatter(
                            stage_v[b], [db_vecs[h], di_vecs[h], col_vec], v
                        )
                return ()

            lax.fori_loop(0, 128, tbody, (), unroll=4)

        def step(u, b, prefetch, launch_next, wait_prev_out):
            wait_gathers(b)
            b1 = 1 - b
            if launch_next:
                wait_idx(b1)
                start_gathers(b1)
            if prefetch:
                start_idx(u + 2, b)
            if wait_prev_out:
                wait_out(b)
            transpose_unit(b)
            start_out(u, b)

        jb0, ib0, jh0 = unit_coords(u0)
        pltpu.sync_copy(
            idx_hbm.at[jb0, ib0, pl.ds(jh0 * _JS * 128, _JS * 128)], idx_v[0]
        )
        start_gathers(0)
        start_idx(u0 + 1, 1)

        step(u0 + 0, 0, prefetch=True, launch_next=True, wait_prev_out=False)
        step(u0 + 1, 1, prefetch=True, launch_next=True, wait_prev_out=False)

        def outer(g, _):
            u = u0 + g * 2
            step(u, 0, prefetch=True, launch_next=True, wait_prev_out=True)
            step(u + 1, 1, prefetch=True, launch_next=True,
                 wait_prev_out=True)
            return ()

        lax.fori_loop(1, _N_OUTER - 1, outer, (), unroll=False)

        u = u0 + (_N_OUTER - 1) * 2
        step(u, 0, prefetch=False, launch_next=True, wait_prev_out=True)
        step(u + 1, 1, prefetch=False, launch_next=False,
             wait_prev_out=True)

        wait_out(0)
        wait_out(1)

    return emb_kernel


def kernel(indices, table):
    idx4 = (
        indices.astype(jnp.int32)
        .T.reshape(_JB, 8, _IB, 128)
        .transpose(0, 2, 1, 3)
        .reshape(_JB, _IB, 8 * 128)
    )
    out5 = _make_sc_lookup()(idx4, table)
    return out5.transpose(2, 4, 0, 1, 3).reshape(_NI, _NJ, _EMB)


# speedup vs baseline: 3.5395x; 1.5584x over previous
import functools

import jax
import jax.numpy as jnp
from jax import lax
from jax.experimental import pallas as pl
from jax.experimental.pallas import tpu as pltpu
from jax.experimental.pallas import tpu_sc as plsc

_NI = 16384
_NJ = 200
_EMB = 32
_JB = _NJ // 8
_IB = _NI // 128
_JS = 4
_N_UNITS = _JB * _IB * 2
_NUM_WORKERS = 32
_PER_WORKER = _N_UNITS // _NUM_WORKERS
_N_OUTER = _PER_WORKER // 2


def _make_sc_lookup():
    mesh = plsc.VectorSubcoreMesh(core_axis_name="c", subcore_axis_name="s")

    _PITCH = 515
    scratch = (
        [pltpu.VMEM((_JS * 128,), jnp.int32) for _ in range(2)]
        + [pltpu.VMEM((_JS * 128, _EMB), jnp.float32) for _ in range(2)]
        + [pltpu.VMEM((4, 8, _PITCH), jnp.float32) for _ in range(2)]
        + [pltpu.SemaphoreType.DMA for _ in range(6)]
    )

    @functools.partial(
        pl.kernel,
        mesh=mesh,
        out_type=jax.ShapeDtypeStruct((_NJ, 4, _IB, 8, 128), jnp.float32),
        scratch_types=scratch,
        compiler_params=pltpu.CompilerParams(
            use_tc_tiling_on_sc=False, needs_layout_passes=False
        ),
    )
    def emb_kernel(idx_hbm, table_hbm, out_hbm, *scr):
        idx_v = scr[0:2]
        rows_v = scr[2:4]
        stage_v = scr[4:6]
        si = scr[6:8]
        sg = scr[8:10]
        so = scr[10:12]

        wid = lax.axis_index("s") * 2 + lax.axis_index("c")
        u0 = wid * _PER_WORKER
        iota16 = lax.iota(jnp.int32, 16)

        def unit_coords(u):
            jh = lax.rem(u, 2)
            ib = lax.rem(lax.div(u, 2), _IB)
            jb = lax.div(u, 2 * _IB)
            return jb, ib, jh

        def start_idx(u, b):
            jb, ib, jh = unit_coords(u)
            pltpu.async_copy(
                idx_hbm.at[jb, ib, pl.ds(jh * _JS * 128, _JS * 128)],
                idx_v[b],
                si[b],
            )

        def wait_idx(b):
            pltpu.make_async_copy(
                idx_hbm.at[0, 0, pl.ds(0, _JS * 128)], idx_v[b], si[b]
            ).wait()

        def start_gathers(b):
            pltpu.async_copy(table_hbm.at[idx_v[b]], rows_v[b], sg[b])

        def wait_gathers(b):
            pltpu.make_async_copy(
                table_hbm.at[idx_v[b]], rows_v[b], sg[b]
            ).wait()

        def start_out(u, b):
            jb, ib, jh = unit_coords(u)
            j0 = jb * 8 + jh * _JS
            for ji in range(_JS):
                pltpu.async_copy(
                    stage_v[b].at[:, :, pl.ds(ji * 128, 128)],
                    out_hbm.at[j0 + ji, :, ib, :, :],
                    so[b],
                )

        def wait_out(b):
            for _ in range(_JS):
                pltpu.make_async_copy(
                    stage_v[b].at[:, :, pl.ds(0, 128)],
                    out_hbm.at[0, :, 0, :, :],
                    so[b],
                ).wait()

        def transpose_unit(b):
            db_vecs = [(iota16 + h * 16) // 8 for h in range(2)]
            di_vecs = [(iota16 + h * 16) % 8 for h in range(2)]

            @plsc.parallel_loop(0, 128, unroll=4)
            def tbody(ii):
                for ji in range(_JS):
                    r = ji * 128 + ii
                    col_vec = jnp.full((16,), r, jnp.int32)
                    for h in range(2):
                        v = rows_v[b][r, pl.ds(h * 16, 16)]
                        plsc.store_scatter(
                            stage_v[b], [db_vecs[h], di_vecs[h], col_vec], v
                        )

        def step(u, b, prefetch, launch_next, wait_prev_out):
            wait_gathers(b)
            b1 = 1 - b
            if launch_next:
                wait_idx(b1)
                start_gathers(b1)
            if prefetch:
                start_idx(u + 2, b)
            if wait_prev_out:
                wait_out(b)
            transpose_unit(b)
            start_out(u, b)

        jb0, ib0, jh0 = unit_coords(u0)
        pltpu.sync_copy(
            idx_hbm.at[jb0, ib0, pl.ds(jh0 * _JS * 128, _JS * 128)], idx_v[0]
        )
        start_gathers(0)
        start_idx(u0 + 1, 1)

        step(u0 + 0, 0, prefetch=True, launch_next=True, wait_prev_out=False)
        step(u0 + 1, 1, prefetch=True, launch_next=True, wait_prev_out=False)

        def outer(g, _):
            u = u0 + g * 2
            step(u, 0, prefetch=True, launch_next=True, wait_prev_out=True)
            step(u + 1, 1, prefetch=True, launch_next=True,
                 wait_prev_out=True)
            return ()

        lax.fori_loop(1, _N_OUTER - 1, outer, (), unroll=False)

        u = u0 + (_N_OUTER - 1) * 2
        step(u, 0, prefetch=False, launch_next=True, wait_prev_out=True)
        step(u + 1, 1, prefetch=False, launch_next=False,
             wait_prev_out=True)

        wait_out(0)
        wait_out(1)

    return emb_kernel


def kernel(indices, table):
    idx4 = (
        indices.astype(jnp.int32)
        .T.reshape(_JB, 8, _IB, 128)
        .transpose(0, 2, 1, 3)
        .reshape(_JB, _IB, 8 * 128)
    )
    out5 = _make_sc_lookup()(idx4, table)
    return out5.transpose(2, 4, 0, 1, 3).reshape(_NI, _NJ, _EMB)
